# baseline (speedup 1.0000x reference)
import functools

import jax
import jax.numpy as jnp
from jax import lax
from jax.experimental import pallas as pl
from jax.experimental.pallas import tpu as pltpu
from jax.experimental.pallas import tpu_sc as plsc

ALPHA = 0.7
TEMP = 2.0
PAD_ID = 0
TOPK = 128

_ROWS_PER_BLOCK = 8
_NUM_WORKERS = 32
_LANES = 16
_NBIN = 256
_UNROLL = 8


def _keys_of(v):
    ti = lax.bitcast_convert_type(v, jnp.int32)
    return ti ^ ((ti >> 31) & jnp.int32(0x7FFFFFFF))


def _sc_scan(hist_ref, b0, rn):
    def cond(st):
        return jnp.logical_not(st[3])

    def body(st):
        b, carry, rr, _ = st
        tot = jnp.sum(hist_ref[pl.ds(b * _LANES, _LANES)])
        within = (carry + tot) >= rr
        return (
            jnp.where(within, b, b - 1),
            jnp.where(within, carry, carry + tot),
            rr,
            within,
        )

    st = lax.while_loop(cond, body, (b0, jnp.int32(0), rn, jnp.bool_(False)))
    return st[0], st[2] - st[1]


def _sc_threshold_body(t_hbm, thr_hbm, row_v, key_v, hist_v, thr_v, *,
                       rows_per_w, vocab, topk):
    nvec = vocab // _LANES
    ones = jnp.ones((_LANES,), jnp.int32)
    zvec = jnp.zeros((_LANES,), jnp.int32)
    lane = lax.broadcasted_iota(jnp.int32, (_LANES,), 0)
    wid = lax.axis_index("s") * 2 + lax.axis_index("c")
    base = wid * rows_per_w

    def row_body(lr, thrvec):
        pltpu.sync_copy(t_hbm.at[base + lr], row_v)

        prefix = jnp.int32(0)
        rn = jnp.int32(topk)
        for level in range(4):
            sh = 24 - 8 * level

            def zbody(j, _):
                for k in range(_UNROLL):
                    hist_v[pl.ds((j * _UNROLL + k) * _LANES, _LANES)] = zvec
                return 0

            lax.fori_loop(0, _NBIN // _UNROLL, zbody, 0)

            def pbody(i, bmax, level=level, sh=sh, prefix=prefix):
                for k in range(_UNROLL):
                    off = (i * _UNROLL + k) * _LANES
                    if level == 0:
                        v = row_v[pl.ds(off, _LANES)]
                        key = _keys_of(v)
                        key_v[pl.ds(off, _LANES)] = key
                        b = (key >> 24) + 128
                        idx = (b << 4) | lane
                        plsc.addupdate_scatter(hist_v, [idx], ones)
                        bmax = jnp.maximum(bmax, b)
                    else:
                        key = key_v[pl.ds(off, _LANES)]
                        m = (key >> (sh + 8)) == prefix
                        b = (key >> sh) & jnp.int32(0xFF)
                        idx = (b << 4) | lane
                        plsc.addupdate_scatter(hist_v, [idx], ones, mask=m)
                        bmax = jnp.maximum(bmax, jnp.where(m, b, 0))
                return bmax

            bmax = lax.fori_loop(0, nvec // _UNROLL, pbody, zvec)
            bin_l, rn = _sc_scan(hist_v, jnp.max(bmax), rn)
            if level == 0:
                prefix = bin_l - 128
            else:
                prefix = (prefix << 8) | bin_l

        thrvec = jnp.where(lane == (lr % _LANES), prefix, thrvec)

        @pl.when(lr % _LANES == _LANES - 1)
        def _flush():
            thr_v[pl.ds((lr // _LANES) * _LANES, _LANES)] = thrvec

        return thrvec

    lax.fori_loop(0, rows_per_w, row_body, zvec)
    pltpu.sync_copy(thr_v, thr_hbm.at[pl.ds(base, rows_per_w)])


def _sc_thresholds(t2):
    n, vocab = t2.shape
    rows_per_w = n // _NUM_WORKERS
    mesh = plsc.VectorSubcoreMesh(core_axis_name="c", subcore_axis_name="s")
    body = functools.partial(
        _sc_threshold_body, rows_per_w=rows_per_w, vocab=vocab, topk=TOPK)
    return pl.kernel(
        body,
        out_type=jax.ShapeDtypeStruct((n,), jnp.int32),
        mesh=mesh,
        compiler_params=pltpu.CompilerParams(needs_layout_passes=False),
        scratch_types=[
            pltpu.VMEM((vocab,), jnp.float32),
            pltpu.VMEM((vocab,), jnp.int32),
            pltpu.VMEM((_NBIN * _LANES,), jnp.int32),
            pltpu.VMEM((rows_per_w,), jnp.int32),
        ],
    )(t2)


def _loss_block_kernel(s_ref, t_ref, lab_ref, am_ref, thr_ref, kl_ref, nm_ref,
                       ce_ref, nv_ref, *, n_rows, temp, pad_id):
    i = pl.program_id(0)
    r = s_ref.shape[0]
    v = s_ref.shape[1]

    s = s_ref[...]
    t = t_ref[...]
    lab = lab_ref[0]
    am = am_ref[0]
    thr = thr_ref[0]

    row_ids = i * r + jax.lax.broadcasted_iota(jnp.int32, (r, 1), 0)
    row_valid = row_ids < n_rows

    inv_t = jnp.float32(1.0 / temp)

    m = jnp.max(s, axis=-1, keepdims=True)
    sm = s - m
    e1 = jnp.exp(sm * inv_t)
    if temp == 2.0:
        e2 = e1 * e1
    else:
        e2 = jnp.exp(sm)
    log_z1 = jnp.log(jnp.sum(e1, axis=-1, keepdims=True))
    log_z2 = jnp.log(jnp.sum(e2, axis=-1, keepdims=True))

    col = jax.lax.broadcasted_iota(jnp.int32, (r, v), 1)
    s_lab = jnp.sum(jnp.where(col == lab, s, 0.0), axis=-1, keepdims=True)
    nll = -(s_lab - m - log_z2)
    valid = (lab != pad_id) & row_valid
    ce_part = jnp.sum(jnp.where(valid, nll, 0.0))
    nv_part = jnp.sum(valid.astype(jnp.float32))

    keep = _keys_of(t) >= thr

    mt = jnp.max(t, axis=-1, keepdims=True)
    tm = (t - mt) * inv_t
    et = jnp.where(keep, jnp.exp(tm), 0.0)
    zt = jnp.sum(et, axis=-1, keepdims=True)
    log_zt = jnp.log(zt)
    log_ps = sm * inv_t - log_z1
    klt = et * (tm - log_zt - log_ps)
    kl_row = jnp.sum(jnp.where(keep, klt, 0.0), axis=-1, keepdims=True) / zt
    rmask = (am != 0) & row_valid
    kl_part = jnp.sum(jnp.where(rmask, kl_row, 0.0))
    nm_part = jnp.sum(rmask.astype(jnp.float32))

    zero = jnp.zeros((1, 1), jnp.float32)

    @pl.when(i == 0)
    def _init():
        kl_ref[...] = zero
        nm_ref[...] = zero
        ce_ref[...] = zero
        nv_ref[...] = zero

    kl_ref[...] = kl_ref[...] + kl_part
    nm_ref[...] = nm_ref[...] + nm_part
    ce_ref[...] = ce_ref[...] + ce_part
    nv_ref[...] = nv_ref[...] + nv_part


def kernel(student_logits, teacher_logits, labels, attention_mask):
    b, s, v = teacher_logits.shape
    n = b * s
    n_rows = b * (s - 1)

    s2 = student_logits.reshape(n, v)
    t2 = teacher_logits.reshape(n, v)
    lab = jnp.concatenate(
        [labels.reshape(n)[1:], jnp.full((1,), PAD_ID, jnp.int32)])
    am = jnp.concatenate(
        [attention_mask.reshape(n)[1:].astype(jnp.int32),
         jnp.zeros((1,), jnp.int32)])

    thr = _sc_thresholds(t2)

    r = _ROWS_PER_BLOCK
    nb = n // r
    lab3 = lab.reshape(nb, r, 1)
    am3 = am.reshape(nb, r, 1)
    thr3 = thr.reshape(nb, r, 1)

    body = functools.partial(
        _loss_block_kernel, n_rows=n_rows, temp=TEMP, pad_id=PAD_ID)

    out_sds = [jax.ShapeDtypeStruct((1, 1), jnp.float32)] * 4
    scalar_spec = pl.BlockSpec((1, 1), lambda i: (0, 0))
    small_spec = pl.BlockSpec((1, r, 1), lambda i: (i, 0, 0))
    kl_sum, nm, ce_sum, nv = pl.pallas_call(
        body,
        grid=(nb,),
        in_specs=[
            pl.BlockSpec((r, v), lambda i: (i, 0)),
            pl.BlockSpec((r, v), lambda i: (i, 0)),
            small_spec,
            small_spec,
            small_spec,
        ],
        out_specs=[scalar_spec] * 4,
        out_shape=out_sds,
    )(s2, t2, lab3, am3, thr3)

    kl = kl_sum[0, 0] / jnp.maximum(nm[0, 0], 1.0) * (TEMP * TEMP)
    ce = ce_sum[0, 0] / jnp.maximum(nv[0, 0], 1.0)
    return ALPHA * kl + (1.0 - ALPHA) * ce


# speedup vs baseline: 4.2582x; 4.2582x over previous
import functools

import jax
import jax.numpy as jnp
from jax.experimental import pallas as pl

ALPHA = 0.7
TEMP = 2.0
PAD_ID = 0
TOPK = 128

_ROWS_PER_BLOCK = 16
_SEARCH_ITERS = 33
_INT_MIN = -(2 ** 31)
_INT_MAX = 2 ** 31 - 1


def _avg_int32(lo, hi):
    return (lo >> 1) + (hi >> 1) + (lo & hi & 1)


def _loss_block_kernel(s_ref, t_ref, lab_ref, am_ref, kl_ref, nm_ref, ce_ref,
                       nv_ref, *, n_rows, temp, topk, pad_id):
    i = pl.program_id(0)
    r = s_ref.shape[0]
    v = s_ref.shape[1]

    s = s_ref[...]
    t = t_ref[...]
    lab = lab_ref[0]
    am = am_ref[0]

    row_ids = i * r + jax.lax.broadcasted_iota(jnp.int32, (r, 1), 0)
    row_valid = row_ids < n_rows

    inv_t = jnp.float32(1.0 / temp)

    m = jnp.max(s, axis=-1, keepdims=True)
    sm = s - m
    e1 = jnp.exp(sm * inv_t)
    if temp == 2.0:
        e2 = e1 * e1
    else:
        e2 = jnp.exp(sm)
    log_z1 = jnp.log(jnp.sum(e1, axis=-1, keepdims=True))
    log_z2 = jnp.log(jnp.sum(e2, axis=-1, keepdims=True))

    col = jax.lax.broadcasted_iota(jnp.int32, (r, v), 1)
    s_lab = jnp.sum(jnp.where(col == lab, s, 0.0), axis=-1, keepdims=True)
    nll = -(s_lab - m - log_z2)
    valid = (lab != pad_id) & row_valid
    ce_part = jnp.sum(jnp.where(valid, nll, 0.0))
    nv_part = jnp.sum(valid.astype(jnp.float32))

    ti = jax.lax.bitcast_convert_type(t, jnp.int32)
    key = ti ^ ((ti >> 31) & jnp.int32(0x7FFFFFFF))

    gm = key
    w = v
    while w > topk:
        w //= 2
        gm = jnp.maximum(gm[:, :w], gm[:, w:2 * w])
    lb = jnp.min(gm, axis=-1, keepdims=True)
    ub = jnp.max(gm, axis=-1, keepdims=True)

    def _cond(carry):
        lo, hi, _ = carry
        return jnp.any(lo <= hi)

    def _body(carry):
        lo, hi, ans = carry
        live = lo <= hi
        mid = _avg_int32(lo, hi)
        cnt = jnp.sum((key >= mid).astype(jnp.int32), axis=-1, keepdims=True)
        eq = (cnt == topk) & live
        ge = (cnt >= topk) & live
        lt = (cnt < topk) & live
        ans = jnp.where(ge, mid, ans)
        lo = jnp.where(eq, jnp.int32(1), jnp.where(ge, mid + 1, lo))
        hi = jnp.where(eq, jnp.int32(0), jnp.where(lt, mid - 1, hi))
        return lo, hi, ans

    _, _, ans = jax.lax.while_loop(_cond, _body, (lb + 1, ub, lb))

    keep = key >= ans

    mt = jnp.max(t, axis=-1, keepdims=True)
    tm = (t - mt) * inv_t
    et = jnp.where(keep, jnp.exp(tm), 0.0)
    zt = jnp.sum(et, axis=-1, keepdims=True)
    log_zt = jnp.log(zt)
    log_ps = sm * inv_t - log_z1
    klt = et * (tm - log_zt - log_ps)
    kl_row = jnp.sum(jnp.where(keep, klt, 0.0), axis=-1, keepdims=True) / zt
    rmask = (am != 0) & row_valid
    kl_part = jnp.sum(jnp.where(rmask, kl_row, 0.0))
    nm_part = jnp.sum(rmask.astype(jnp.float32))

    zero = jnp.zeros((1, 1), jnp.float32)

    @pl.when(i == 0)
    def _init():
        kl_ref[...] = zero
        nm_ref[...] = zero
        ce_ref[...] = zero
        nv_ref[...] = zero

    kl_ref[...] = kl_ref[...] + kl_part
    nm_ref[...] = nm_ref[...] + nm_part
    ce_ref[...] = ce_ref[...] + ce_part
    nv_ref[...] = nv_ref[...] + nv_part


def kernel(student_logits, teacher_logits, labels, attention_mask):
    b, s, v = teacher_logits.shape
    n = b * s
    n_rows = b * (s - 1)

    s2 = student_logits.reshape(n, v)
    t2 = teacher_logits.reshape(n, v)
    lab = jnp.concatenate(
        [labels.reshape(n)[1:], jnp.full((1,), PAD_ID, jnp.int32)])
    am = jnp.concatenate(
        [attention_mask.reshape(n)[1:].astype(jnp.int32),
         jnp.zeros((1,), jnp.int32)])

    r = _ROWS_PER_BLOCK
    nb = n // r
    lab3 = lab.reshape(nb, r, 1)
    am3 = am.reshape(nb, r, 1)

    body = functools.partial(
        _loss_block_kernel, n_rows=n_rows, temp=TEMP, topk=TOPK, pad_id=PAD_ID)

    out_sds = [jax.ShapeDtypeStruct((1, 1), jnp.float32)] * 4
    scalar_spec = pl.BlockSpec((1, 1), lambda i: (0, 0))
    kl_sum, nm, ce_sum, nv = pl.pallas_call(
        body,
        grid=(nb,),
        in_specs=[
            pl.BlockSpec((r, v), lambda i: (i, 0)),
            pl.BlockSpec((r, v), lambda i: (i, 0)),
            pl.BlockSpec((1, r, 1), lambda i: (i, 0, 0)),
            pl.BlockSpec((1, r, 1), lambda i: (i, 0, 0)),
        ],
        out_specs=[scalar_spec] * 4,
        out_shape=out_sds,
    )(s2, t2, lab3, am3)

    kl = kl_sum[0, 0] / jnp.maximum(nm[0, 0], 1.0) * (TEMP * TEMP)
    ce = ce_sum[0, 0] / jnp.maximum(nv[0, 0], 1.0)
    return ALPHA * kl + (1.0 - ALPHA) * ce


# speedup vs baseline: 4.7545x; 1.1166x over previous
import functools

import jax
import jax.numpy as jnp
from jax.experimental import pallas as pl

ALPHA = 0.7
TEMP = 2.0
PAD_ID = 0
TOPK = 128

_ROWS_PER_BLOCK = 32
_SEARCH_ITERS = 33
_INT_MIN = -(2 ** 31)
_INT_MAX = 2 ** 31 - 1


def _avg_int32(lo, hi):
    return (lo >> 1) + (hi >> 1) + (lo & hi & 1)


def _loss_block_kernel(s_ref, t_ref, lab_ref, am_ref, kl_ref, nm_ref, ce_ref,
                       nv_ref, *, n_rows, temp, topk, pad_id):
    i = pl.program_id(0)
    r = s_ref.shape[0]
    v = s_ref.shape[1]

    s = s_ref[...]
    t = t_ref[...]
    lab = lab_ref[0]
    am = am_ref[0]

    row_ids = i * r + jax.lax.broadcasted_iota(jnp.int32, (r, 1), 0)
    row_valid = row_ids < n_rows

    inv_t = jnp.float32(1.0 / temp)

    m = jnp.max(s, axis=-1, keepdims=True)
    sm = s - m
    e1 = jnp.exp(sm * inv_t)
    if temp == 2.0:
        e2 = e1 * e1
    else:
        e2 = jnp.exp(sm)
    log_z1 = jnp.log(jnp.sum(e1, axis=-1, keepdims=True))
    log_z2 = jnp.log(jnp.sum(e2, axis=-1, keepdims=True))

    col = jax.lax.broadcasted_iota(jnp.int32, (r, v), 1)
    s_lab = jnp.sum(jnp.where(col == lab, s, 0.0), axis=-1, keepdims=True)
    nll = -(s_lab - m - log_z2)
    valid = (lab != pad_id) & row_valid
    ce_part = jnp.sum(jnp.where(valid, nll, 0.0))
    nv_part = jnp.sum(valid.astype(jnp.float32))

    ti = jax.lax.bitcast_convert_type(t, jnp.int32)
    key = ti ^ ((ti >> 31) & jnp.int32(0x7FFFFFFF))

    gm = key
    w = v
    while w > topk:
        w //= 2
        gm = jnp.maximum(gm[:, :w], gm[:, w:2 * w])
    lb = jnp.min(gm, axis=-1, keepdims=True)
    ub = jnp.max(gm, axis=-1, keepdims=True)

    def _cond(carry):
        lo, hi, _ = carry
        return jnp.any(lo <= hi)

    def _body(carry):
        lo, hi, ans = carry
        live = lo <= hi
        mid = _avg_int32(lo, hi)
        cnt = jnp.sum((key >= mid).astype(jnp.int32), axis=-1, keepdims=True)
        eq = (cnt == topk) & live
        ge = (cnt >= topk) & live
        lt = (cnt < topk) & live
        ans = jnp.where(ge, mid, ans)
        lo = jnp.where(eq, jnp.int32(1), jnp.where(ge, mid + 1, lo))
        hi = jnp.where(eq, jnp.int32(0), jnp.where(lt, mid - 1, hi))
        return lo, hi, ans

    _, _, ans = jax.lax.while_loop(_cond, _body, (lb + 1, ub, lb))

    keep = key >= ans

    mt = jnp.max(t, axis=-1, keepdims=True)
    tm = (t - mt) * inv_t
    et = jnp.where(keep, jnp.exp(tm), 0.0)
    zt = jnp.sum(et, axis=-1, keepdims=True)
    log_zt = jnp.log(zt)
    log_ps = sm * inv_t - log_z1
    klt = et * (tm - log_zt - log_ps)
    kl_row = jnp.sum(jnp.where(keep, klt, 0.0), axis=-1, keepdims=True) / zt
    rmask = (am != 0) & row_valid
    kl_part = jnp.sum(jnp.where(rmask, kl_row, 0.0))
    nm_part = jnp.sum(rmask.astype(jnp.float32))

    zero = jnp.zeros((1, 1), jnp.float32)

    @pl.when(i == 0)
    def _init():
        kl_ref[...] = zero
        nm_ref[...] = zero
        ce_ref[...] = zero
        nv_ref[...] = zero

    kl_ref[...] = kl_ref[...] + kl_part
    nm_ref[...] = nm_ref[...] + nm_part
    ce_ref[...] = ce_ref[...] + ce_part
    nv_ref[...] = nv_ref[...] + nv_part


def kernel(student_logits, teacher_logits, labels, attention_mask):
    b, s, v = teacher_logits.shape
    n = b * s
    n_rows = b * (s - 1)

    s2 = student_logits.reshape(n, v)
    t2 = teacher_logits.reshape(n, v)
    lab = jnp.concatenate(
        [labels.reshape(n)[1:], jnp.full((1,), PAD_ID, jnp.int32)])
    am = jnp.concatenate(
        [attention_mask.reshape(n)[1:].astype(jnp.int32),
         jnp.zeros((1,), jnp.int32)])

    r = _ROWS_PER_BLOCK
    nb = n // r
    lab3 = lab.reshape(nb, r, 1)
    am3 = am.reshape(nb, r, 1)

    body = functools.partial(
        _loss_block_kernel, n_rows=n_rows, temp=TEMP, topk=TOPK, pad_id=PAD_ID)

    out_sds = [jax.ShapeDtypeStruct((1, 1), jnp.float32)] * 4
    scalar_spec = pl.BlockSpec((1, 1), lambda i: (0, 0))
    kl_sum, nm, ce_sum, nv = pl.pallas_call(
        body,
        grid=(nb,),
        in_specs=[
            pl.BlockSpec((r, v), lambda i: (i, 0)),
            pl.BlockSpec((r, v), lambda i: (i, 0)),
            pl.BlockSpec((1, r, 1), lambda i: (i, 0, 0)),
            pl.BlockSpec((1, r, 1), lambda i: (i, 0, 0)),
        ],
        out_specs=[scalar_spec] * 4,
        out_shape=out_sds,
    )(s2, t2, lab3, am3)

    kl = kl_sum[0, 0] / jnp.maximum(nm[0, 0], 1.0) * (TEMP * TEMP)
    ce = ce_sum[0, 0] / jnp.maximum(nv[0, 0], 1.0)
    return ALPHA * kl + (1.0 - ALPHA) * ce


# speedup vs baseline: 4.9089x; 1.0325x over previous
import functools

import jax
import jax.numpy as jnp
from jax.experimental import pallas as pl

ALPHA = 0.7
TEMP = 2.0
PAD_ID = 0
TOPK = 128

_ROWS_PER_BLOCK = 32
_SEARCH_ITERS = 33
_INT_MIN = -(2 ** 31)
_INT_MAX = 2 ** 31 - 1


def _avg_int32(lo, hi):
    return (lo >> 1) + (hi >> 1) + (lo & hi & 1)


def _loss_block_kernel(s_ref, t_ref, lab_ref, am_ref, kl_ref, nm_ref, ce_ref,
                       nv_ref, *, n_rows, temp, topk, pad_id):
    i = pl.program_id(0)
    r = s_ref.shape[0]
    v = s_ref.shape[1]

    s = s_ref[...]
    t = t_ref[...]
    lab = lab_ref[0]
    am = am_ref[0]

    row_ids = i * r + jax.lax.broadcasted_iota(jnp.int32, (r, 1), 0)
    row_valid = row_ids < n_rows

    inv_t = jnp.float32(1.0 / temp)

    m = jnp.max(s, axis=-1, keepdims=True)
    sm = s - m
    e1 = jnp.exp(sm * inv_t)
    if temp == 2.0:
        e2 = e1 * e1
    else:
        e2 = jnp.exp(sm)
    log_z1 = jnp.log(jnp.sum(e1, axis=-1, keepdims=True))
    log_z2 = jnp.log(jnp.sum(e2, axis=-1, keepdims=True))

    col = jax.lax.broadcasted_iota(jnp.int32, (r, v), 1)
    s_lab = jnp.sum(jnp.where(col == lab, s, 0.0), axis=-1, keepdims=True)
    nll = -(s_lab - m - log_z2)
    valid = (lab != pad_id) & row_valid
    ce_part = jnp.sum(jnp.where(valid, nll, 0.0))
    nv_part = jnp.sum(valid.astype(jnp.float32))

    ti = jax.lax.bitcast_convert_type(t, jnp.int32)
    key = ti ^ ((ti >> 31) & jnp.int32(0x7FFFFFFF))

    gm = key
    w = v
    while w > topk:
        w //= 2
        gm = jnp.maximum(gm[:, :w], gm[:, w:2 * w])
    lb = jnp.min(gm, axis=-1, keepdims=True)
    ub = jnp.max(gm, axis=-1, keepdims=True)

    log_topk = jnp.float32(jnp.log(float(topk)))

    def _cond(carry):
        return jnp.any(carry[0] <= carry[1])

    def _body(carry):
        lo, hi, ans, mid, px, plog, it = carry
        live = lo <= hi
        cnt = jnp.sum((key >= mid).astype(jnp.int32), axis=-1, keepdims=True)
        logc = jnp.log(cnt.astype(jnp.float32))
        eq = (cnt == topk) & live
        ge = (cnt >= topk) & live
        lt = (cnt < topk) & live
        ans = jnp.where(ge, mid, ans)
        lo = jnp.where(eq, jnp.int32(1), jnp.where(ge, mid + 1, lo))
        hi = jnp.where(eq, jnp.int32(0), jnp.where(lt, mid - 1, hi))
        bis = _avg_int32(lo, hi)
        midf = mid.astype(jnp.float32)
        denom = midf - px.astype(jnp.float32)
        slope = (logc - plog) / denom
        sec_f = jnp.clip(midf + (log_topk - logc) / slope,
                         lo.astype(jnp.float32), hi.astype(jnp.float32))
        use_sec = (denom != 0) & (slope < 0) & (it % 3 != 0)
        sec = jnp.clip(sec_f.astype(jnp.int32), lo, hi)
        nmid = jnp.where(use_sec, sec, bis)
        return lo, hi, ans, nmid, mid, logc, it + 1

    lo0 = lb + 1
    st = (lo0, ub, lb, _avg_int32(lo0, ub), lb,
          jnp.full_like(lb, jnp.log(float(v)), dtype=jnp.float32),
          jnp.int32(1))
    ans = jax.lax.while_loop(_cond, _body, st)[2]

    keep = key >= ans

    mt = jnp.max(t, axis=-1, keepdims=True)
    tm = (t - mt) * inv_t
    et = jnp.where(keep, jnp.exp(tm), 0.0)
    zt = jnp.sum(et, axis=-1, keepdims=True)
    log_zt = jnp.log(zt)
    log_ps = sm * inv_t - log_z1
    klt = et * (tm - log_zt - log_ps)
    kl_row = jnp.sum(jnp.where(keep, klt, 0.0), axis=-1, keepdims=True) / zt
    rmask = (am != 0) & row_valid
    kl_part = jnp.sum(jnp.where(rmask, kl_row, 0.0))
    nm_part = jnp.sum(rmask.astype(jnp.float32))

    zero = jnp.zeros((1, 1), jnp.float32)

    @pl.when(i == 0)
    def _init():
        kl_ref[...] = zero
        nm_ref[...] = zero
        ce_ref[...] = zero
        nv_ref[...] = zero

    kl_ref[...] = kl_ref[...] + kl_part
    nm_ref[...] = nm_ref[...] + nm_part
    ce_ref[...] = ce_ref[...] + ce_part
    nv_ref[...] = nv_ref[...] + nv_part


def kernel(student_logits, teacher_logits, labels, attention_mask):
    b, s, v = teacher_logits.shape
    n = b * s
    n_rows = b * (s - 1)

    s2 = student_logits.reshape(n, v)
    t2 = teacher_logits.reshape(n, v)
    lab = jnp.concatenate(
        [labels.reshape(n)[1:], jnp.full((1,), PAD_ID, jnp.int32)])
    am = jnp.concatenate(
        [attention_mask.reshape(n)[1:].astype(jnp.int32),
         jnp.zeros((1,), jnp.int32)])

    r = _ROWS_PER_BLOCK
    nb = n // r
    lab3 = lab.reshape(nb, r, 1)
    am3 = am.reshape(nb, r, 1)

    body = functools.partial(
        _loss_block_kernel, n_rows=n_rows, temp=TEMP, topk=TOPK, pad_id=PAD_ID)

    out_sds = [jax.ShapeDtypeStruct((1, 1), jnp.float32)] * 4
    scalar_spec = pl.BlockSpec((1, 1), lambda i: (0, 0))
    kl_sum, nm, ce_sum, nv = pl.pallas_call(
        body,
        grid=(nb,),
        in_specs=[
            pl.BlockSpec((r, v), lambda i: (i, 0)),
            pl.BlockSpec((r, v), lambda i: (i, 0)),
            pl.BlockSpec((1, r, 1), lambda i: (i, 0, 0)),
            pl.BlockSpec((1, r, 1), lambda i: (i, 0, 0)),
        ],
        out_specs=[scalar_spec] * 4,
        out_shape=out_sds,
    )(s2, t2, lab3, am3)

    kl = kl_sum[0, 0] / jnp.maximum(nm[0, 0], 1.0) * (TEMP * TEMP)
    ce = ce_sum[0, 0] / jnp.maximum(nv[0, 0], 1.0)
    return ALPHA * kl + (1.0 - ALPHA) * ce
